# Initial kernel scaffold; baseline (speedup 1.0000x reference)
import jax
import jax.numpy as jnp
from jax.experimental import pallas as pl


def kernel(inputs, W):
    raise NotImplementedError("write your pallas kernel here")


# speedup vs baseline: 2.3196x; 2.3196x over previous
import jax
import jax.numpy as jnp
from jax.experimental import pallas as pl


BLK = 256


def _bcast_kernel(w_ref, out_ref):
    w = w_ref[...]
    out_ref[...] = jnp.broadcast_to(w[:, None, :], out_ref.shape)


def kernel(inputs, W):
    seq_len, b_sz = inputs.shape
    emb = W.shape[1]
    grid = (seq_len // BLK,)
    return pl.pallas_call(
        _bcast_kernel,
        grid=grid,
        in_specs=[pl.BlockSpec((BLK, emb), lambda i: (i, 0))],
        out_specs=pl.BlockSpec((BLK, b_sz, emb), lambda i: (i, 0, 0)),
        out_shape=jax.ShapeDtypeStruct((seq_len, b_sz, emb), W.dtype),
    )(W[:seq_len])


# speedup vs baseline: 2.5262x; 1.0891x over previous
import jax
import jax.numpy as jnp
from jax.experimental import pallas as pl


BLK = 512


def _bcast_kernel(w_ref, out_ref):
    w = w_ref[...]
    out_ref[...] = jnp.broadcast_to(w[:, None, :], out_ref.shape)


def kernel(inputs, W):
    seq_len, b_sz = inputs.shape
    emb = W.shape[1]
    grid = (seq_len // BLK,)
    return pl.pallas_call(
        _bcast_kernel,
        grid=grid,
        in_specs=[pl.BlockSpec((BLK, emb), lambda i: (i, 0))],
        out_specs=pl.BlockSpec((BLK, b_sz, emb), lambda i: (i, 0, 0)),
        out_shape=jax.ShapeDtypeStruct((seq_len, b_sz, emb), W.dtype),
    )(W[:seq_len])


# speedup vs baseline: 2.5673x; 1.0163x over previous
import jax
import jax.numpy as jnp
from jax.experimental import pallas as pl


BLK = 1024


def _bcast_kernel(w_ref, out_ref):
    w = w_ref[...]
    out_ref[...] = jnp.broadcast_to(w[:, None, :], out_ref.shape)


def kernel(inputs, W):
    seq_len, b_sz = inputs.shape
    emb = W.shape[1]
    grid = (seq_len // BLK,)
    return pl.pallas_call(
        _bcast_kernel,
        grid=grid,
        in_specs=[pl.BlockSpec((BLK, emb), lambda i: (i, 0))],
        out_specs=pl.BlockSpec((BLK, b_sz, emb), lambda i: (i, 0, 0)),
        out_shape=jax.ShapeDtypeStruct((seq_len, b_sz, emb), W.dtype),
    )(W[:seq_len])
